# baseline (speedup 1.0000x reference)
import functools

import jax
import jax.numpy as jnp
from jax import lax
from jax.experimental import pallas as pl
from jax.experimental.pallas import tpu as pltpu
from jax.experimental.pallas import tpu_sc as plsc

_B = 16384
_NC, _NS = 2, 16
_NW = _NC * _NS
_RPW = _B // _NW
_CHUNK = 128
_NCHUNK = _RPW // _CHUNK
_NPAR = 16
_NCH = 8
_W = _NPAR + _NPAR * _NCH
_WPAD = 256

_mesh = plsc.VectorSubcoreMesh(core_axis_name="c", subcore_axis_name="s")


@functools.partial(
    pl.kernel,
    mesh=_mesh,
    compiler_params=pltpu.CompilerParams(
        needs_layout_passes=False, use_tc_tiling_on_sc=True
    ),
    out_type=jax.ShapeDtypeStruct((_NW, _NPAR), jnp.float32),
    scratch_types=[
        pltpu.VMEM((_CHUNK, _WPAD), jnp.float32),
        pltpu.VMEM((_CHUNK, _WPAD), jnp.float32),
        pltpu.VMEM((_NPAR,), jnp.float32),
        pltpu.SemaphoreType.DMA,
        pltpu.SemaphoreType.DMA,
    ],
)
def _sc_penalty(x_hbm, out_hbm, buf0, buf1, part, sem0, sem1):
    wid = lax.axis_index("s") * _NC + lax.axis_index("c")
    base = wid * _RPW
    bufs = [buf0, buf1]
    sems = [sem0, sem1]

    colbase = lax.iota(jnp.int32, _NPAR) * _NCH + _NPAR
    cols = [colbase + c for c in range(_NCH)]

    def start_copy(k):
        return pltpu.async_copy(
            x_hbm.at[pl.ds(base + k * _CHUNK, _CHUNK), pl.ds(0, _WPAD)],
            bufs[k % 2],
            sems[k % 2],
        )

    acc = jnp.zeros((_NPAR,), jnp.float32)
    copies = [start_copy(0)]
    for k in range(_NCHUNK):
        if k + 1 < _NCHUNK:
            copies.append(start_copy(k + 1))
        copies[k].wait()
        buf = bufs[k % 2]

        @plsc.parallel_loop(0, _CHUNK, carry=acc)
        def row_term(r, a, buf=buf):
            rowv = jnp.full((_NPAR,), r, jnp.int32)
            m = plsc.load_gather(buf, [rowv, cols[0]])
            for c in range(1, _NCH):
                m = jnp.maximum(m, plsc.load_gather(buf, [rowv, cols[c]]))
            p = buf[r, pl.ds(0, _NPAR)]
            return a + jnp.abs(p - m)

        acc = row_term

    part[...] = acc
    pltpu.sync_copy(part, out_hbm.at[wid])


def kernel(preds):
    partials = _sc_penalty(preds)
    return jnp.sum(partials) / preds.shape[0]


# speedup vs baseline: 2.6915x; 2.6915x over previous
import functools

import jax
import jax.numpy as jnp
from jax import lax
from jax.experimental import pallas as pl
from jax.experimental.pallas import tpu as pltpu
from jax.experimental.pallas import tpu_sc as plsc

_B = 16384
_NC, _NS = 2, 16
_NW = _NC * _NS
_CPW = _B // _NW
_CCH = 256
_NCHUNK = _CPW // _CCH
_W = 144
_NPAR = 16
_NCH = 8
_L = 16

_mesh = plsc.VectorSubcoreMesh(core_axis_name="c", subcore_axis_name="s")


@functools.partial(
    pl.kernel,
    mesh=_mesh,
    compiler_params=pltpu.CompilerParams(needs_layout_passes=False),
    out_type=jax.ShapeDtypeStruct((_NW, _L), jnp.float32),
    scratch_types=[
        pltpu.VMEM((_W, _CCH), jnp.float32),
        pltpu.VMEM((_W, _CCH), jnp.float32),
        pltpu.VMEM((_L,), jnp.float32),
        pltpu.SemaphoreType.DMA,
        pltpu.SemaphoreType.DMA,
    ],
)
def _sc_penalty(pt_hbm, out_hbm, buf0, buf1, acc_ref, sem0, sem1):
    wid = lax.axis_index("s") * _NC + lax.axis_index("c")
    base = wid * _CPW
    bufs = [buf0, buf1]
    sems = [sem0, sem1]

    def start_copy(k):
        return pltpu.async_copy(
            pt_hbm.at[pl.ds(0, _W), pl.ds(base + k * _CCH, _CCH)],
            bufs[k % 2],
            sems[k % 2],
        )

    acc = jnp.zeros((_L,), jnp.float32)
    copies = [start_copy(0)]
    for k in range(_NCHUNK):
        if k + 1 < _NCHUNK:
            copies.append(start_copy(k + 1))
        copies[k].wait()
        buf = bufs[k % 2]

        def group_body(g, a, buf=buf):
            sl = pl.ds(g * _L, _L)
            t = None
            for p in range(_NPAR):
                m = buf[_NPAR + _NCH * p, sl]
                for c in range(1, _NCH):
                    m = jnp.maximum(m, buf[_NPAR + _NCH * p + c, sl])
                d = jnp.abs(buf[p, sl] - m)
                t = d if t is None else t + d
            return a + t

        acc = lax.fori_loop(0, _CCH // _L, group_body, acc)

    acc_ref[...] = acc
    pltpu.sync_copy(acc_ref, out_hbm.at[wid])


def kernel(preds):
    partials = _sc_penalty(preds.T)
    return jnp.sum(partials) / preds.shape[0]
